# baseline (speedup 1.0000x reference)
import functools

import jax
import jax.numpy as jnp
from jax import lax
from jax.experimental import pallas as pl
from jax.experimental.pallas import tpu as pltpu
from jax.experimental.pallas import tpu_sc as plsc

L = 16
NC, NS = 2, 16
NW = NC * NS
CHUNK = 128


def _neg_sqrt(x):
    xi = plsc.bitcast(x, jnp.int32)
    y = plsc.bitcast(jnp.int32(0x5F3759DF) - (xi >> 1), jnp.float32)
    for _ in range(3):
        y = y * (1.5 - 0.5 * x * y * y)
    return -(x * y)


def _transe_body(heads_hbm, rels_hbm, tails_hbm, ent_hbm, rel_hbm, out_hbm,
                 hidx_v, ridx_v, tidx_v, h_v, r_v, t_v, out_v,
                 hsem, rsem, tsem, *, rows_per_worker, emb_dim):
    wid = lax.axis_index("s") * NC + lax.axis_index("c")
    base = wid * rows_per_worker
    n_chunks = rows_per_worker // CHUNK
    groups = CHUNK // L

    for k in range(n_chunks):
        off = base + k * CHUNK
        pltpu.sync_copy(heads_hbm.at[pl.ds(off, CHUNK)], hidx_v.at[k])
        pltpu.sync_copy(rels_hbm.at[pl.ds(off, CHUNK)], ridx_v.at[k])
        pltpu.sync_copy(tails_hbm.at[pl.ds(off, CHUNK)], tidx_v.at[k])
        ch = pltpu.async_copy(ent_hbm.at[hidx_v.at[k]], h_v, hsem)
        cr = pltpu.async_copy(rel_hbm.at[ridx_v.at[k]], r_v, rsem)
        ct = pltpu.async_copy(ent_hbm.at[tidx_v.at[k]], t_v, tsem)
        ch.wait()
        cr.wait()
        ct.wait()

        def group_body(i, _, k=k):
            rows = i * L + lax.iota(jnp.int32, 16)
            col0 = jnp.zeros((16,), jnp.int32)
            acc = jnp.zeros((16,), jnp.float32)
            for j in range(emb_dim):
                col = col0 + j
                h = plsc.load_gather(h_v, [rows, col])
                r = plsc.load_gather(r_v, [rows, col])
                t = plsc.load_gather(t_v, [rows, col])
                d = h + r - t
                acc = acc + d * d
            out_v[pl.ds(k * CHUNK + i * L, L)] = _neg_sqrt(acc)
            return 0

        lax.fori_loop(0, groups, group_body, 0)

    pltpu.sync_copy(out_v, out_hbm.at[pl.ds(base, rows_per_worker)])


def kernel(heads, rels, tails, ent_embeds, rel_embeds):
    batch = heads.shape[0]
    emb_dim = ent_embeds.shape[1]
    rows_per_worker = batch // NW

    mesh = plsc.VectorSubcoreMesh(core_axis_name="c", subcore_axis_name="s")
    body = functools.partial(_transe_body, rows_per_worker=rows_per_worker,
                             emb_dim=emb_dim)
    n_chunks = rows_per_worker // CHUNK
    run = pl.kernel(
        body,
        out_type=jax.ShapeDtypeStruct((batch,), jnp.float32),
        mesh=mesh,
        compiler_params=pltpu.CompilerParams(needs_layout_passes=False,
                                             use_tc_tiling_on_sc=False),
        scratch_types=[
            pltpu.VMEM((n_chunks, CHUNK), jnp.int32),
            pltpu.VMEM((n_chunks, CHUNK), jnp.int32),
            pltpu.VMEM((n_chunks, CHUNK), jnp.int32),
            pltpu.VMEM((CHUNK, emb_dim), jnp.float32),
            pltpu.VMEM((CHUNK, emb_dim), jnp.float32),
            pltpu.VMEM((CHUNK, emb_dim), jnp.float32),
            pltpu.VMEM((rows_per_worker,), jnp.float32),
            pltpu.SemaphoreType.DMA,
            pltpu.SemaphoreType.DMA,
            pltpu.SemaphoreType.DMA,
        ],
    )
    return run(heads, rels, tails, ent_embeds, rel_embeds)


# speedup vs baseline: 1.0118x; 1.0118x over previous
import functools

import jax
import jax.numpy as jnp
from jax import lax
from jax.experimental import pallas as pl
from jax.experimental.pallas import tpu as pltpu
from jax.experimental.pallas import tpu_sc as plsc

L = 16
NC, NS = 2, 16
NW = NC * NS
CHUNK = 128


def _neg_sqrt(x):
    xi = plsc.bitcast(x, jnp.int32)
    y = plsc.bitcast(jnp.int32(0x5F3759DF) - (xi >> 1), jnp.float32)
    for _ in range(3):
        y = y * (1.5 - 0.5 * x * y * y)
    return -(x * y)


def _transe_body(heads_hbm, rels_hbm, tails_hbm, ent_hbm, rel_hbm, out_hbm,
                 hidx_v, ridx_v, tidx_v, h_v, r_v, t_v, out_v,
                 isem, hsem, rsem, tsem, *, rows_per_worker, emb_dim):
    wid = lax.axis_index("s") * NC + lax.axis_index("c")
    base = wid * rows_per_worker
    n_chunks = rows_per_worker // CHUNK
    groups = CHUNK // L

    ic = []
    for k in range(n_chunks):
        off = base + k * CHUNK
        ic.append(pltpu.async_copy(
            heads_hbm.at[pl.ds(off, CHUNK)], hidx_v.at[k], isem))
        ic.append(pltpu.async_copy(
            rels_hbm.at[pl.ds(off, CHUNK)], ridx_v.at[k], isem))
        ic.append(pltpu.async_copy(
            tails_hbm.at[pl.ds(off, CHUNK)], tidx_v.at[k], isem))
    for c in ic:
        c.wait()

    gc = []
    for k in range(n_chunks):
        gc.append(pltpu.async_copy(ent_hbm.at[hidx_v.at[k]], h_v.at[k], hsem))
        gc.append(pltpu.async_copy(rel_hbm.at[ridx_v.at[k]], r_v.at[k], rsem))
        gc.append(pltpu.async_copy(ent_hbm.at[tidx_v.at[k]], t_v.at[k], tsem))

    for k in range(n_chunks):
        gc[3 * k].wait()
        gc[3 * k + 1].wait()
        gc[3 * k + 2].wait()

        def group_body(i, _, k=k):
            rows = i * L + lax.iota(jnp.int32, 16)
            col0 = jnp.zeros((16,), jnp.int32)
            acc = jnp.zeros((16,), jnp.float32)
            for j in range(emb_dim):
                col = col0 + j
                h = plsc.load_gather(h_v.at[k], [rows, col])
                r = plsc.load_gather(r_v.at[k], [rows, col])
                t = plsc.load_gather(t_v.at[k], [rows, col])
                d = h + r - t
                acc = acc + d * d
            out_v[pl.ds(k * CHUNK + i * L, L)] = _neg_sqrt(acc)
            return 0

        lax.fori_loop(0, groups, group_body, 0)

    pltpu.sync_copy(out_v, out_hbm.at[pl.ds(base, rows_per_worker)])


def kernel(heads, rels, tails, ent_embeds, rel_embeds):
    batch = heads.shape[0]
    emb_dim = ent_embeds.shape[1]
    rows_per_worker = batch // NW

    mesh = plsc.VectorSubcoreMesh(core_axis_name="c", subcore_axis_name="s")
    body = functools.partial(_transe_body, rows_per_worker=rows_per_worker,
                             emb_dim=emb_dim)
    n_chunks = rows_per_worker // CHUNK
    run = pl.kernel(
        body,
        out_type=jax.ShapeDtypeStruct((batch,), jnp.float32),
        mesh=mesh,
        compiler_params=pltpu.CompilerParams(needs_layout_passes=False,
                                             use_tc_tiling_on_sc=False),
        scratch_types=[
            pltpu.VMEM((n_chunks, CHUNK), jnp.int32),
            pltpu.VMEM((n_chunks, CHUNK), jnp.int32),
            pltpu.VMEM((n_chunks, CHUNK), jnp.int32),
            pltpu.VMEM((n_chunks, CHUNK, emb_dim), jnp.float32),
            pltpu.VMEM((n_chunks, CHUNK, emb_dim), jnp.float32),
            pltpu.VMEM((n_chunks, CHUNK, emb_dim), jnp.float32),
            pltpu.VMEM((rows_per_worker,), jnp.float32),
            pltpu.SemaphoreType.DMA,
            pltpu.SemaphoreType.DMA,
            pltpu.SemaphoreType.DMA,
            pltpu.SemaphoreType.DMA,
        ],
    )
    return run(heads, rels, tails, ent_embeds, rel_embeds)


# speedup vs baseline: 1.0244x; 1.0124x over previous
import functools

import jax
import jax.numpy as jnp
from jax import lax
from jax.experimental import pallas as pl
from jax.experimental.pallas import tpu as pltpu
from jax.experimental.pallas import tpu_sc as plsc

L = 16
NC, NS = 2, 16
NW = NC * NS
CHUNK = 128
PACKW = 128


def _neg_sqrt(x):
    xi = plsc.bitcast(x, jnp.int32)
    y = plsc.bitcast(jnp.int32(0x5F3759DF) - (xi >> 1), jnp.float32)
    for _ in range(3):
        y = y * (1.5 - 0.5 * x * y * y)
    return -(x * y)


def _transe_body(heads_hbm, rels_hbm, tails_hbm, ent_hbm, rel_hbm, out_hbm,
                 hidx_v, ridx_v, tidx_v, hrow_v, trow_v, h_v, r_v, t_v, out_v,
                 isem, hsem, rsem, tsem, *, rows_per_worker, emb_dim):
    wid = lax.axis_index("s") * NC + lax.axis_index("c")
    base = wid * rows_per_worker
    n_chunks = rows_per_worker // CHUNK
    groups = CHUNK // L
    vecs = CHUNK // L

    ic = []
    for k in range(n_chunks):
        off = base + k * CHUNK
        ic.append(pltpu.async_copy(
            heads_hbm.at[pl.ds(off, CHUNK)], hidx_v.at[k], isem))
        ic.append(pltpu.async_copy(
            rels_hbm.at[pl.ds(off, CHUNK)], ridx_v.at[k], isem))
        ic.append(pltpu.async_copy(
            tails_hbm.at[pl.ds(off, CHUNK)], tidx_v.at[k], isem))
    for c in ic:
        c.wait()

    for k in range(n_chunks):
        for v in range(vecs):
            hrow_v.at[k][pl.ds(v * L, L)] = hidx_v.at[k][pl.ds(v * L, L)] >> 1
            trow_v.at[k][pl.ds(v * L, L)] = tidx_v.at[k][pl.ds(v * L, L)] >> 1

    rc = [pltpu.async_copy(rel_hbm.at[ridx_v.at[k]], r_v.at[k], rsem)
          for k in range(n_chunks)]
    hc, tc = [], []

    def fire(k):
        s = k % 2
        hc.append(pltpu.async_copy(ent_hbm.at[hrow_v.at[k]], h_v.at[s], hsem))
        tc.append(pltpu.async_copy(ent_hbm.at[trow_v.at[k]], t_v.at[s], tsem))

    fire(0)
    if n_chunks > 1:
        fire(1)

    lanes = lax.iota(jnp.int32, L)

    for k in range(n_chunks):
        s = k % 2
        hc[k].wait()
        tc[k].wait()
        rc[k].wait()

        def group_body(i, _, k=k, s=s):
            rows = i * L + lanes
            hpar = (hidx_v.at[k][pl.ds(i * L, L)] & 1) * emb_dim
            tpar = (tidx_v.at[k][pl.ds(i * L, L)] & 1) * emb_dim
            col0 = jnp.zeros((L,), jnp.int32)
            acc = jnp.zeros((L,), jnp.float32)
            for j in range(emb_dim):
                h = plsc.load_gather(h_v.at[s], [rows, hpar + j])
                r = plsc.load_gather(r_v.at[k], [rows, col0 + j])
                t = plsc.load_gather(t_v.at[s], [rows, tpar + j])
                d = h + r - t
                acc = acc + d * d
            out_v[pl.ds(k * CHUNK + i * L, L)] = _neg_sqrt(acc)
            return 0

        lax.fori_loop(0, groups, group_body, 0)
        if k + 2 < n_chunks:
            fire(k + 2)

    pltpu.sync_copy(out_v, out_hbm.at[pl.ds(base, rows_per_worker)])


def kernel(heads, rels, tails, ent_embeds, rel_embeds):
    batch = heads.shape[0]
    ent_num, emb_dim = ent_embeds.shape
    rows_per_worker = batch // NW
    pack = PACKW // emb_dim
    ent_packed = lax.optimization_barrier(ent_embeds.reshape(ent_num // pack, PACKW))

    mesh = plsc.VectorSubcoreMesh(core_axis_name="c", subcore_axis_name="s")
    body = functools.partial(_transe_body, rows_per_worker=rows_per_worker,
                             emb_dim=emb_dim)
    n_chunks = rows_per_worker // CHUNK
    run = pl.kernel(
        body,
        out_type=jax.ShapeDtypeStruct((batch,), jnp.float32),
        mesh=mesh,
        compiler_params=pltpu.CompilerParams(needs_layout_passes=False,
                                             use_tc_tiling_on_sc=False),
        scratch_types=[
            pltpu.VMEM((n_chunks, CHUNK), jnp.int32),
            pltpu.VMEM((n_chunks, CHUNK), jnp.int32),
            pltpu.VMEM((n_chunks, CHUNK), jnp.int32),
            pltpu.VMEM((n_chunks, CHUNK), jnp.int32),
            pltpu.VMEM((n_chunks, CHUNK), jnp.int32),
            pltpu.VMEM((2, CHUNK, PACKW), jnp.float32),
            pltpu.VMEM((n_chunks, CHUNK, emb_dim), jnp.float32),
            pltpu.VMEM((2, CHUNK, PACKW), jnp.float32),
            pltpu.VMEM((rows_per_worker,), jnp.float32),
            pltpu.SemaphoreType.DMA,
            pltpu.SemaphoreType.DMA,
            pltpu.SemaphoreType.DMA,
            pltpu.SemaphoreType.DMA,
        ],
    )
    return run(heads, rels, tails, ent_packed, rel_embeds)
